# baseline (speedup 1.0000x reference)
import functools

import jax
import jax.numpy as jnp
from jax import lax
from jax.experimental import pallas as pl
from jax.experimental.pallas import tpu as pltpu
from jax.experimental.pallas import tpu_sc as plsc

_LANES = 16
_BLK = 128

_DNUMS = lax.GatherDimensionNumbers(
    offset_dims=(), collapsed_slice_dims=(0,), start_index_map=(0,))


def _lane_splat(vec, idx):
    return lax.gather(vec, idx[:, None], _DNUMS, slice_sizes=(1,),
                      mode=lax.GatherScatterMode.PROMISE_IN_BOUNDS)


def _gather_body(n, n_j, d, x_hbm, idx_hbm, out_hbm, idx_v, blk_v, col_v):
    cid = lax.axis_index("c")
    sid = lax.axis_index("s")
    lane = lax.iota(jnp.int32, _LANES)

    for k in range(n):

        @pl.when(jnp.logical_and(cid == 0, sid == k))
        def _(k=k):
            pltpu.sync_copy(idx_hbm, idx_v)
            hk = idx_v[...][k]
            wk = idx_v[...][n + k % n_j]
            ht = pl.multiple_of((hk // _BLK) * _BLK, _BLK)
            l = hk - ht
            pltpu.sync_copy(x_hbm.at[wk, :, pl.ds(ht, _BLK)], blk_v)
            b16 = pl.multiple_of((l // _LANES) * _LANES, _LANES)
            lvec = jnp.broadcast_to(l - b16, (_LANES,)).astype(jnp.int32)
            for g in range(d // _LANES):
                acc = jnp.zeros((_LANES,), jnp.float32)
                for c16 in range(_LANES):
                    v16 = blk_v[g * _LANES + c16, pl.ds(b16, _LANES)]
                    acc = jnp.where(lane == c16, _lane_splat(v16, lvec), acc)
                col_v[0, pl.ds(g * _LANES, _LANES)] = acc
            pltpu.sync_copy(col_v, out_hbm.at[k])


def kernel(x, index1, index2):
    h, w, d = x.shape
    n_i, n_j = index1.shape
    n = n_i * n_j
    idx = jnp.concatenate([index1.reshape(-1).astype(jnp.int32),
                           index2.astype(jnp.int32)])
    idx = jnp.pad(idx, (0, _LANES - n - n_j))

    mesh = plsc.VectorSubcoreMesh(core_axis_name="c", subcore_axis_name="s",
                                  num_cores=1)
    run = pl.kernel(
        functools.partial(_gather_body, n, n_j, d),
        out_type=jax.ShapeDtypeStruct((_LANES, 8, d), jnp.float32),
        mesh=mesh,
        scratch_types=[
            pltpu.VMEM((_LANES,), jnp.int32),
            pltpu.VMEM((d, _BLK), jnp.float32),
            pltpu.VMEM((8, d), jnp.float32),
        ],
    )
    out16 = run(jnp.transpose(x, (1, 2, 0)), idx)
    return out16[:n, 0, :].reshape(n_i, n_j, d)


# speedup vs baseline: 1.0372x; 1.0372x over previous
import functools

import jax
import jax.numpy as jnp
from jax import lax
from jax.experimental import pallas as pl
from jax.experimental.pallas import tpu as pltpu
from jax.experimental.pallas import tpu_sc as plsc

_LANES = 16
_BLK = 128

_DNUMS = lax.GatherDimensionNumbers(
    offset_dims=(), collapsed_slice_dims=(0,), start_index_map=(0,))


def _lane_splat(vec, idx):
    return lax.gather(vec, idx[:, None], _DNUMS, slice_sizes=(1,),
                      mode=lax.GatherScatterMode.PROMISE_IN_BOUNDS)


def _gather_body(n, n_j, d, x_hbm, idx_hbm, out_hbm, idx_v, blk_v, col_v):
    cid = lax.axis_index("c")
    sid = lax.axis_index("s")
    lane = lax.iota(jnp.int32, _LANES)

    @pl.when(jnp.logical_and(cid == 0, sid < n))
    def _():
        pltpu.sync_copy(idx_hbm, idx_v)
        vals = idx_v[...]
        hk = vals[0]
        wk = vals[n]
        for k in range(1, n):
            hk = jnp.where(sid == k, vals[k], hk)
        for j in range(1, n_j):
            wk = jnp.where(sid % n_j == j, vals[n + j], wk)
        ht = pl.multiple_of((hk // _BLK) * _BLK, _BLK)
        l = hk - ht
        pltpu.sync_copy(x_hbm.at[wk, :, pl.ds(ht, _BLK)], blk_v)
        b16 = pl.multiple_of((l // _LANES) * _LANES, _LANES)
        lvec = jnp.broadcast_to(l - b16, (_LANES,)).astype(jnp.int32)
        for g in range(d // _LANES):
            acc = jnp.zeros((_LANES,), jnp.float32)
            for c16 in range(_LANES):
                v16 = blk_v[g * _LANES + c16, pl.ds(b16, _LANES)]
                acc = jnp.where(lane == c16, _lane_splat(v16, lvec), acc)
            col_v[0, pl.ds(g * _LANES, _LANES)] = acc
        pltpu.sync_copy(col_v, out_hbm.at[sid])


def kernel(x, index1, index2):
    h, w, d = x.shape
    n_i, n_j = index1.shape
    n = n_i * n_j
    idx = jnp.concatenate([index1.reshape(-1).astype(jnp.int32),
                           index2.astype(jnp.int32)])
    idx = jnp.pad(idx, (0, _LANES - n - n_j))

    mesh = plsc.VectorSubcoreMesh(core_axis_name="c", subcore_axis_name="s")
    run = pl.kernel(
        functools.partial(_gather_body, n, n_j, d),
        out_type=jax.ShapeDtypeStruct((_LANES, 8, d), jnp.float32),
        mesh=mesh,
        scratch_types=[
            pltpu.VMEM((_LANES,), jnp.int32),
            pltpu.VMEM((d, _BLK), jnp.float32),
            pltpu.VMEM((8, d), jnp.float32),
        ],
    )
    out16 = run(jnp.transpose(x, (1, 2, 0)), idx)
    return out16[:n, 0, :].reshape(n_i, n_j, d)
